# baseline (speedup 1.0000x reference)
import functools

import jax
import jax.numpy as jnp
from jax import lax
from jax.experimental import pallas as pl
from jax.experimental.pallas import tpu as pltpu
from jax.experimental.pallas import tpu_sc as plsc

LANES = 16


def _sc_gather_body(n_dim, c_dim, l_dim, paths_per_w, h_dim, n_edges,
                    table_hbm, paths_hbm, base_hbm, out_hbm,
                    nodes_v, idx_v, base_v, rows_v, acc_v,
                    sem2, sg0, sg1, sg2, sg3):
    wid = lax.axis_index("s") * 2 + lax.axis_index("c")
    nodes_per_w = paths_per_w * l_dim
    rows_per_w = nodes_per_w

    cp_n = pltpu.async_copy(paths_hbm.at[pl.ds(wid * nodes_per_w, nodes_per_w)],
                            nodes_v.at[pl.ds(0, nodes_per_w)], sem2)
    cp_b = pltpu.async_copy(base_hbm.at[wid], base_v, sem2)
    cp_n.wait()
    cp_b.wait()
    nodes_v[pl.ds(nodes_per_w, LANES)] = jnp.zeros((LANES,), jnp.int32)
    base_vec = base_v[...]

    n_chunks = nodes_per_w // LANES

    def idx_body(t, carry):
        uu = nodes_v[pl.ds(t * LANES, LANES)]
        vv = nodes_v[pl.ds(t * LANES + 1, LANES)]
        idx_v[pl.ds(t * LANES, LANES)] = (uu * n_dim + vv) * c_dim + base_vec
        return carry

    lax.fori_loop(0, n_chunks, idx_body, 0)

    q = rows_per_w // 4
    pq = paths_per_w // 4
    gsems = (sg0, sg1, sg2, sg3)
    cps = [pltpu.async_copy(table_hbm.at[idx_v.at[pl.ds(i * q, q)]],
                            rows_v.at[pl.ds(i * q, q)], gsems[i])
           for i in range(4)]

    inv = 1.0 / float(n_edges)
    col_chunks = h_dim // LANES

    def pool_one(j, carry):
        r0 = j * l_dim
        for k in range(col_chunks):
            cs = pl.ds(k * LANES, LANES)
            acc = rows_v[r0, cs]
            for e in range(1, n_edges):
                acc = acc + rows_v[r0 + e, cs]
            acc_v[j, cs] = acc * inv
        return carry

    for i in range(4):
        cps[i].wait()
        lax.fori_loop(i * pq, (i + 1) * pq, pool_one, 0)

    pltpu.sync_copy(acc_v, out_hbm.at[pl.ds(wid * paths_per_w, paths_per_w)])


def _sc_gather(table, paths_flat, base_rows, n_dim, c_dim, l_dim, n_edges):
    n_paths = paths_flat.shape[0] // l_dim
    h_dim = table.shape[1]
    nw = 32
    paths_per_w = n_paths // nw
    rows_per_w = paths_per_w * l_dim

    mesh = plsc.VectorSubcoreMesh(core_axis_name="c", subcore_axis_name="s")
    body = functools.partial(_sc_gather_body, n_dim, c_dim, l_dim,
                             paths_per_w, h_dim, n_edges)
    f = pl.kernel(
        body,
        mesh=mesh,
        out_type=jax.ShapeDtypeStruct((n_paths, h_dim), jnp.float32),
        scratch_types=[
            pltpu.VMEM((rows_per_w + LANES,), jnp.int32),
            pltpu.VMEM((rows_per_w,), jnp.int32),
            pltpu.VMEM((LANES,), jnp.int32),
            pltpu.VMEM((rows_per_w, h_dim), jnp.float32),
            pltpu.VMEM((paths_per_w, h_dim), jnp.float32),
            pltpu.SemaphoreType.DMA,
            pltpu.SemaphoreType.DMA,
            pltpu.SemaphoreType.DMA,
            pltpu.SemaphoreType.DMA,
            pltpu.SemaphoreType.DMA,
        ],
    )
    return f(table, paths_flat, base_rows)


def _tc_mlp_body(b_dim, p_dim, pf_ref, g_ref, w1_ref, b1_ref,
                 w2_ref, b2_ref, mask_ref, probs_ref, logp_ref, ent_ref):
    h_dim = g_ref.shape[1]
    pf = pf_ref[...]
    w1a = w1_ref[0:h_dim, :]
    w1b = w1_ref[h_dim:2 * h_dim, :]
    h1 = jnp.dot(pf, w1a, preferred_element_type=jnp.float32)
    hg = jnp.dot(g_ref[...], w1b, preferred_element_type=jnp.float32)
    h = h1.reshape(b_dim, p_dim, -1) + hg[:, None, :] + b1_ref[...][None, None, :]
    h = jnp.maximum(h, 0.0)
    s = jnp.dot(h.reshape(b_dim * p_dim, -1), w2_ref[...],
                preferred_element_type=jnp.float32)
    s = s.reshape(b_dim, p_dim) + b2_ref[...]
    m = mask_ref[...] > 0.0
    s = jnp.where(m, s, -jnp.inf)
    mx = jnp.max(s, axis=1, keepdims=True)
    e = jnp.exp(s - mx)
    denom = jnp.sum(e, axis=1, keepdims=True)
    probs = e / denom
    logp = s - mx - jnp.log(denom)
    probs_ref[...] = probs
    logp_ref[...] = logp
    ent_ref[...] = -jnp.sum(probs * jnp.where(m, logp, 0.0), axis=1)


def _tc_mlp(path_feat, g, w1, b1, w2, b2, mask_f):
    b_dim, p_dim = mask_f.shape
    body = functools.partial(_tc_mlp_body, b_dim, p_dim)
    return pl.pallas_call(
        body,
        out_shape=[
            jax.ShapeDtypeStruct((b_dim, p_dim), jnp.float32),
            jax.ShapeDtypeStruct((b_dim, p_dim), jnp.float32),
            jax.ShapeDtypeStruct((b_dim,), jnp.float32),
        ],
    )(path_feat, g, w1, b1, w2, b2, mask_f)


def kernel(edge_features, graph_embedding, selected_commodity, candidate_paths,
           path_mask, W1, b1, W2, b2):
    B, N, _, C, H = edge_features.shape
    P, L = candidate_paths.shape[1], candidate_paths.shape[2]
    n_edges = L - 1

    table = edge_features.reshape(B * N * N * C, H)
    paths_flat = candidate_paths.reshape(-1).astype(jnp.int32)
    base = (jnp.arange(32, dtype=jnp.int32) // 2) * (N * N * C) \
        + selected_commodity.astype(jnp.int32)[jnp.arange(32) // 2]
    base_rows = jnp.broadcast_to(base[:, None], (32, LANES))

    path_feat = _sc_gather(table, paths_flat, base_rows, N, C, L, n_edges)
    probs, logp, ent = _tc_mlp(path_feat, graph_embedding, W1, b1, W2, b2,
                               path_mask.astype(jnp.float32))
    return probs, logp, ent


# speedup vs baseline: 1.2600x; 1.2600x over previous
import functools

import jax
import jax.numpy as jnp
from jax import lax
from jax.experimental import pallas as pl
from jax.experimental.pallas import tpu as pltpu
from jax.experimental.pallas import tpu_sc as plsc

LANES = 16


def _sc_gather_body(n_dim, c_dim, l_dim, paths_per_w, h_dim, n_edges,
                    table_hbm, paths_hbm, base_hbm, out_hbm,
                    nodes_v, idx_v, base_v, rows_v, acc_v,
                    sem2, sg0, sg1):
    wid = lax.axis_index("s") * 2 + lax.axis_index("c")
    nodes_per_w = paths_per_w * l_dim
    rows_per_w = nodes_per_w

    cp_n = pltpu.async_copy(paths_hbm.at[pl.ds(wid * nodes_per_w, nodes_per_w)],
                            nodes_v.at[pl.ds(0, nodes_per_w)], sem2)
    cp_b = pltpu.async_copy(base_hbm.at[wid], base_v, sem2)
    cp_n.wait()
    cp_b.wait()
    nodes_v[pl.ds(nodes_per_w, LANES)] = jnp.zeros((LANES,), jnp.int32)
    base_vec = base_v[...]

    n_chunks = nodes_per_w // LANES

    def idx_body(t, carry):
        uu = nodes_v[pl.ds(t * LANES, LANES)]
        vv = nodes_v[pl.ds(t * LANES + 1, LANES)]
        idx_v[pl.ds(t * LANES, LANES)] = (uu * n_dim + vv) * c_dim + base_vec
        return carry

    lax.fori_loop(0, n_chunks, idx_body, 0)

    q = rows_per_w // 2
    pq = paths_per_w // 2
    gsems = (sg0, sg1)
    cps = [pltpu.async_copy(table_hbm.at[idx_v.at[pl.ds(i * q, q)]],
                            rows_v.at[pl.ds(i * q, q)], gsems[i])
           for i in range(2)]

    inv = 1.0 / float(n_edges)
    col_chunks = h_dim // LANES

    for i in range(2):
        cps[i].wait()

        @plsc.parallel_loop(i * pq * col_chunks, (i + 1) * pq * col_chunks,
                            1, unroll=4)
        def pool_body(t):
            j = t >> 4
            k = t & (col_chunks - 1)
            r0 = j * l_dim
            cs = pl.ds(k * LANES, LANES)
            acc = rows_v[r0, cs]
            for e in range(1, n_edges):
                acc = acc + rows_v[r0 + e, cs]
            acc_v[j, cs] = acc * inv

    pltpu.sync_copy(acc_v, out_hbm.at[pl.ds(wid * paths_per_w, paths_per_w)])


def _sc_gather(table, paths_flat, base_rows, n_dim, c_dim, l_dim, n_edges):
    n_paths = paths_flat.shape[0] // l_dim
    h_dim = table.shape[1]
    nw = 32
    paths_per_w = n_paths // nw
    rows_per_w = paths_per_w * l_dim

    mesh = plsc.VectorSubcoreMesh(core_axis_name="c", subcore_axis_name="s")
    body = functools.partial(_sc_gather_body, n_dim, c_dim, l_dim,
                             paths_per_w, h_dim, n_edges)
    f = pl.kernel(
        body,
        mesh=mesh,
        out_type=jax.ShapeDtypeStruct((n_paths, h_dim), jnp.float32),
        scratch_types=[
            pltpu.VMEM((rows_per_w + LANES,), jnp.int32),
            pltpu.VMEM((rows_per_w,), jnp.int32),
            pltpu.VMEM((LANES,), jnp.int32),
            pltpu.VMEM((rows_per_w, h_dim), jnp.float32),
            pltpu.VMEM((paths_per_w, h_dim), jnp.float32),
            pltpu.SemaphoreType.DMA,
            pltpu.SemaphoreType.DMA,
            pltpu.SemaphoreType.DMA,
        ],
    )
    return f(table, paths_flat, base_rows)


def _tc_mlp_body(b_dim, p_dim, pf_ref, g_ref, w1_ref, b1_ref,
                 w2_ref, b2_ref, mask_ref, probs_ref, logp_ref, ent_ref):
    h_dim = g_ref.shape[1]
    pf = pf_ref[...]
    w1a = w1_ref[0:h_dim, :]
    w1b = w1_ref[h_dim:2 * h_dim, :]
    h1 = jnp.dot(pf, w1a, preferred_element_type=jnp.float32)
    hg = jnp.dot(g_ref[...], w1b, preferred_element_type=jnp.float32)
    h = h1.reshape(b_dim, p_dim, -1) + hg[:, None, :] + b1_ref[...][None, None, :]
    h = jnp.maximum(h, 0.0)
    s = jnp.dot(h.reshape(b_dim * p_dim, -1), w2_ref[...],
                preferred_element_type=jnp.float32)
    s = s.reshape(b_dim, p_dim) + b2_ref[...]
    m = mask_ref[...] > 0.0
    s = jnp.where(m, s, -jnp.inf)
    mx = jnp.max(s, axis=1, keepdims=True)
    e = jnp.exp(s - mx)
    denom = jnp.sum(e, axis=1, keepdims=True)
    probs = e / denom
    logp = s - mx - jnp.log(denom)
    probs_ref[...] = probs
    logp_ref[...] = logp
    ent_ref[...] = -jnp.sum(probs * jnp.where(m, logp, 0.0), axis=1)


def _tc_mlp(path_feat, g, w1, b1, w2, b2, mask_f):
    b_dim, p_dim = mask_f.shape
    body = functools.partial(_tc_mlp_body, b_dim, p_dim)
    return pl.pallas_call(
        body,
        out_shape=[
            jax.ShapeDtypeStruct((b_dim, p_dim), jnp.float32),
            jax.ShapeDtypeStruct((b_dim, p_dim), jnp.float32),
            jax.ShapeDtypeStruct((b_dim,), jnp.float32),
        ],
    )(path_feat, g, w1, b1, w2, b2, mask_f)


def kernel(edge_features, graph_embedding, selected_commodity, candidate_paths,
           path_mask, W1, b1, W2, b2):
    B, N, _, C, H = edge_features.shape
    P, L = candidate_paths.shape[1], candidate_paths.shape[2]
    n_edges = L - 1

    table = edge_features.reshape(B * N * N * C, H)
    paths_flat = candidate_paths.reshape(-1).astype(jnp.int32)
    base = (jnp.arange(32, dtype=jnp.int32) // 2) * (N * N * C) \
        + selected_commodity.astype(jnp.int32)[jnp.arange(32) // 2]
    base_rows = jnp.broadcast_to(base[:, None], (32, LANES))

    path_feat = _sc_gather(table, paths_flat, base_rows, N, C, L, n_edges)
    probs, logp, ent = _tc_mlp(path_feat, graph_embedding, W1, b1, W2, b2,
                               path_mask.astype(jnp.float32))
    return probs, logp, ent
